# baseline (speedup 1.0000x reference)
import jax
import jax.numpy as jnp
from jax import lax
from jax.experimental import pallas as pl
from jax.experimental.pallas import tpu as pltpu
from jax.experimental.pallas import tpu_sc as plsc

D_MODEL = 64
BATCH = 4096
SEQ = 200
NUM_WORKERS = 32
B_PER_W = BATCH // NUM_WORKERS


def _emb_body(x_hbm, table_hbm, out_hbm, idx_v, rows_v, sem):
    wid = lax.axis_index("s") * 2 + lax.axis_index("c")
    base = wid * B_PER_W

    def body(i, carry):
        b = base + i
        pltpu.sync_copy(x_hbm.at[b], idx_v)
        pltpu.async_copy(table_hbm.at[idx_v], rows_v, sem).wait()
        pltpu.sync_copy(rows_v, out_hbm.at[b])
        return carry

    lax.fori_loop(0, B_PER_W, body, 0)


@jax.jit
def kernel(x, table):
    mesh = plsc.VectorSubcoreMesh(core_axis_name="c", subcore_axis_name="s")
    f = pl.kernel(
        _emb_body,
        out_type=jax.ShapeDtypeStruct((BATCH, SEQ, D_MODEL), jnp.float32),
        mesh=mesh,
        scratch_types=[
            pltpu.VMEM((SEQ,), jnp.int32),
            pltpu.VMEM((SEQ, D_MODEL), jnp.float32),
            pltpu.SemaphoreType.DMA,
        ],
        compiler_params=pltpu.CompilerParams(use_tc_tiling_on_sc=False),
    )
    return f(x.astype(jnp.int32), table)


# speedup vs baseline: 1.7801x; 1.7801x over previous
import jax
import jax.numpy as jnp
from jax import lax
from jax.experimental import pallas as pl
from jax.experimental.pallas import tpu as pltpu
from jax.experimental.pallas import tpu_sc as plsc

VOCAB = 1000000
D_MODEL = 64
BATCH = 4096
SEQ = 200
B_TOTAL = BATCH * SEQ

NUM_WORKERS = 32
B_PER_W = B_TOTAL // NUM_WORKERS
CHUNK = 512
N_CHUNKS = B_PER_W // CHUNK

VB = 16384
A_GRID = (VOCAB + VB - 1) // VB

BB = 128
C_GRID = BATCH // BB


def _detile_body(t_ref, out_ref):
    z = t_ref[...].T
    out_ref[...] = jnp.concatenate([z[: VB // 2], z[VB // 2:]], axis=1)


def _gather_body(x_hbm, table_hbm, out_hbm, idx_v, rows_v, sem):
    wid = lax.axis_index("s") * 2 + lax.axis_index("c")
    base = wid * B_PER_W

    def body(i, carry):
        off = base + i * CHUNK
        pltpu.sync_copy(x_hbm.at[pl.ds(off, CHUNK)], idx_v)
        pltpu.async_copy(table_hbm.at[idx_v], rows_v, sem).wait()
        pltpu.sync_copy(rows_v, out_hbm.at[pl.ds(off, CHUNK)])
        return carry

    lax.fori_loop(0, N_CHUNKS, body, 0)


def _retile_body(in_ref, out_ref):
    out_ref[...] = in_ref[...].T


@jax.jit
def kernel(x, table):
    packed = pl.pallas_call(
        _detile_body,
        grid=(A_GRID,),
        in_specs=[pl.BlockSpec((D_MODEL, VB), lambda i: (0, i))],
        out_specs=pl.BlockSpec((VB // 2, 2 * D_MODEL), lambda i: (i, 0)),
        out_shape=jax.ShapeDtypeStruct((A_GRID * VB // 2, 2 * D_MODEL),
                                       jnp.float32),
    )(table.T)
    table_l = packed.reshape(-1).reshape(A_GRID * VB, D_MODEL)

    xf = x.reshape(-1).astype(jnp.int32)
    half = VB // 2
    xf = 2 * ((xf // VB) * half + xf % half) + (xf // half) % 2
    mesh = plsc.VectorSubcoreMesh(core_axis_name="c", subcore_axis_name="s")
    rows = pl.kernel(
        _gather_body,
        out_type=jax.ShapeDtypeStruct((B_TOTAL, D_MODEL), jnp.float32),
        mesh=mesh,
        scratch_types=[
            pltpu.VMEM((CHUNK,), jnp.int32),
            pltpu.VMEM((CHUNK, D_MODEL), jnp.float32),
            pltpu.SemaphoreType.DMA,
        ],
        compiler_params=pltpu.CompilerParams(use_tc_tiling_on_sc=False),
    )(xf, table_l)

    out_t = pl.pallas_call(
        _retile_body,
        grid=(C_GRID,),
        in_specs=[pl.BlockSpec((BB, SEQ * D_MODEL), lambda i: (i, 0))],
        out_specs=pl.BlockSpec((SEQ * D_MODEL, BB), lambda i: (0, i)),
        out_shape=jax.ShapeDtypeStruct((SEQ * D_MODEL, BATCH), jnp.float32),
    )(rows.reshape(BATCH, SEQ * D_MODEL))
    return out_t.reshape(SEQ, D_MODEL, BATCH).transpose(2, 0, 1)


# speedup vs baseline: 2.2628x; 1.2712x over previous
import jax
import jax.numpy as jnp
from jax import lax
from jax.experimental import pallas as pl
from jax.experimental.pallas import tpu as pltpu
from jax.experimental.pallas import tpu_sc as plsc

VOCAB = 1000000
D_MODEL = 64
BATCH = 4096
SEQ = 200
B_TOTAL = BATCH * SEQ

NUM_WORKERS = 32
B_PER_W = B_TOTAL // NUM_WORKERS
CHUNK = 512
N_CHUNKS = B_PER_W // CHUNK

VB = 16384
A_GRID = (VOCAB + VB - 1) // VB

BB = 128
C_GRID = BATCH // BB


def _detile_body(t_ref, out_ref):
    z = t_ref[...].T
    out_ref[...] = jnp.concatenate([z[: VB // 2], z[VB // 2:]], axis=1)


def _gather_body(x_hbm, oidx_hbm, table_hbm, out_hbm, idx_v, oidx_v, rows_v,
                 sem, osem):
    wid = lax.axis_index("s") * 2 + lax.axis_index("c")
    base = wid * B_PER_W

    def body(i, carry):
        off = base + i * CHUNK
        pltpu.sync_copy(x_hbm.at[pl.ds(off, CHUNK)], idx_v)
        pltpu.sync_copy(oidx_hbm.at[pl.ds(off, CHUNK)], oidx_v)
        pltpu.async_copy(table_hbm.at[idx_v], rows_v, sem).wait()
        pltpu.async_copy(rows_v, out_hbm.at[oidx_v], osem).wait()
        return carry

    lax.fori_loop(0, N_CHUNKS, body, 0)


def _retile_body(in_ref, out_ref):
    y = in_ref[...]
    out_ref[...] = y.transpose(0, 2, 1).reshape(SEQ * D_MODEL, BB)


@jax.jit
def kernel(x, table):
    packed = pl.pallas_call(
        _detile_body,
        grid=(A_GRID,),
        in_specs=[pl.BlockSpec((D_MODEL, VB), lambda i: (0, i))],
        out_specs=pl.BlockSpec((VB // 2, 2 * D_MODEL), lambda i: (i, 0)),
        out_shape=jax.ShapeDtypeStruct((A_GRID * VB // 2, 2 * D_MODEL),
                                       jnp.float32),
    )(table.T)
    table_l = packed.reshape(-1).reshape(A_GRID * VB, D_MODEL)

    xf = x.reshape(-1).astype(jnp.int32)
    hb = 13
    xf = ((xf >> (hb + 1)) << (hb + 1)) + ((xf & (VB // 2 - 1)) << 1) + (
        (xf >> hb) & 1)
    bb = jax.lax.broadcasted_iota(jnp.int32, (BATCH, SEQ), 0)
    tt = jax.lax.broadcasted_iota(jnp.int32, (BATCH, SEQ), 1)
    oidx = (((tt >> 1) * BATCH + bb) * 2 + (tt & 1)).reshape(-1)
    mesh = plsc.VectorSubcoreMesh(core_axis_name="c", subcore_axis_name="s")
    rows = pl.kernel(
        _gather_body,
        out_type=jax.ShapeDtypeStruct((B_TOTAL, D_MODEL), jnp.float32),
        mesh=mesh,
        scratch_types=[
            pltpu.VMEM((CHUNK,), jnp.int32),
            pltpu.VMEM((CHUNK,), jnp.int32),
            pltpu.VMEM((CHUNK, D_MODEL), jnp.float32),
            pltpu.SemaphoreType.DMA,
            pltpu.SemaphoreType.DMA,
        ],
        compiler_params=pltpu.CompilerParams(use_tc_tiling_on_sc=False),
    )(xf, oidx, table_l)

    out_t = pl.pallas_call(
        _retile_body,
        grid=(C_GRID,),
        in_specs=[pl.BlockSpec((SEQ // 2, BB, 2 * D_MODEL),
                               lambda i: (0, i, 0))],
        out_specs=pl.BlockSpec((SEQ * D_MODEL, BB), lambda i: (0, i)),
        out_shape=jax.ShapeDtypeStruct((SEQ * D_MODEL, BATCH), jnp.float32),
    )(rows.reshape(SEQ // 2, BATCH, 2 * D_MODEL))
    return out_t.reshape(SEQ, D_MODEL, BATCH).transpose(2, 0, 1)


# speedup vs baseline: 2.3153x; 1.0232x over previous
import jax
import jax.numpy as jnp
from jax import lax
from jax.experimental import pallas as pl
from jax.experimental.pallas import tpu as pltpu
from jax.experimental.pallas import tpu_sc as plsc

VOCAB = 1000000
D_MODEL = 64
BATCH = 4096
SEQ = 200
B_TOTAL = BATCH * SEQ

NUM_WORKERS = 32
B_PER_W = B_TOTAL // NUM_WORKERS
CHUNK = 512
N_CHUNKS = B_PER_W // CHUNK

VB = 32768
A_GRID = (VOCAB + VB - 1) // VB

BB = 128
C_GRID = BATCH // BB


def _detile_body(t_ref, out_ref):
    z = t_ref[...].T
    out_ref[...] = jnp.concatenate([z[: VB // 2], z[VB // 2:]], axis=1)


def _gather_body(x_hbm, oidx_hbm, table_hbm, out_hbm, idx_v, oidx_v, rows_v,
                 sem, osem):
    wid = lax.axis_index("s") * 2 + lax.axis_index("c")
    base = wid * B_PER_W

    def body(i, carry):
        off = base + i * CHUNK
        pltpu.sync_copy(x_hbm.at[pl.ds(off, CHUNK)], idx_v)
        pltpu.sync_copy(oidx_hbm.at[pl.ds(off, CHUNK)], oidx_v)
        pltpu.async_copy(table_hbm.at[idx_v], rows_v, sem).wait()
        pltpu.async_copy(rows_v, out_hbm.at[oidx_v], osem).wait()
        return carry

    lax.fori_loop(0, N_CHUNKS, body, 0)


def _retile_body(in_ref, out_ref):
    y = in_ref[...]
    out_ref[...] = y.transpose(0, 2, 1).reshape(SEQ * D_MODEL, BB)


@jax.jit
def kernel(x, table):
    packed = pl.pallas_call(
        _detile_body,
        grid=(A_GRID,),
        in_specs=[pl.BlockSpec((D_MODEL, VB), lambda i: (0, i))],
        out_specs=pl.BlockSpec((VB // 2, 2 * D_MODEL), lambda i: (i, 0)),
        out_shape=jax.ShapeDtypeStruct((A_GRID * VB // 2, 2 * D_MODEL),
                                       jnp.float32),
    )(table.T)
    table_l = packed.reshape(-1).reshape(A_GRID * VB, D_MODEL)

    xf = x.reshape(-1).astype(jnp.int32)
    hb = (VB // 2).bit_length() - 1
    xf = ((xf >> (hb + 1)) << (hb + 1)) + ((xf & (VB // 2 - 1)) << 1) + (
        (xf >> hb) & 1)
    bb = jax.lax.broadcasted_iota(jnp.int32, (BATCH, SEQ), 0)
    tt = jax.lax.broadcasted_iota(jnp.int32, (BATCH, SEQ), 1)
    oidx = (((tt >> 1) * BATCH + bb) * 2 + (tt & 1)).reshape(-1)
    mesh = plsc.VectorSubcoreMesh(core_axis_name="c", subcore_axis_name="s")
    rows = pl.kernel(
        _gather_body,
        out_type=jax.ShapeDtypeStruct((B_TOTAL, D_MODEL), jnp.float32),
        mesh=mesh,
        scratch_types=[
            pltpu.VMEM((CHUNK,), jnp.int32),
            pltpu.VMEM((CHUNK,), jnp.int32),
            pltpu.VMEM((CHUNK, D_MODEL), jnp.float32),
            pltpu.SemaphoreType.DMA,
            pltpu.SemaphoreType.DMA,
        ],
        compiler_params=pltpu.CompilerParams(use_tc_tiling_on_sc=False),
    )(xf, oidx, table_l)

    out_t = pl.pallas_call(
        _retile_body,
        grid=(C_GRID,),
        in_specs=[pl.BlockSpec((SEQ // 2, BB, 2 * D_MODEL),
                               lambda i: (0, i, 0))],
        out_specs=pl.BlockSpec((SEQ * D_MODEL, BB), lambda i: (0, i)),
        out_shape=jax.ShapeDtypeStruct((SEQ * D_MODEL, BATCH), jnp.float32),
    )(rows.reshape(SEQ // 2, BATCH, 2 * D_MODEL))
    return out_t.reshape(SEQ, D_MODEL, BATCH).transpose(2, 0, 1)


# speedup vs baseline: 2.5902x; 1.1187x over previous
import jax
import jax.numpy as jnp
from jax import lax
from jax.experimental import pallas as pl
from jax.experimental.pallas import tpu as pltpu
from jax.experimental.pallas import tpu_sc as plsc

VOCAB = 1000000
D_MODEL = 64
BATCH = 4096
SEQ = 200
B_TOTAL = BATCH * SEQ

NUM_WORKERS = 32
B_PER_W = B_TOTAL // NUM_WORKERS
CHUNK = 256
N_CHUNKS = B_PER_W // CHUNK
NBUF = 4
N_OUTER = N_CHUNKS // NBUF

VB = 32768
A_GRID = (VOCAB + VB - 1) // VB

BB = 128
C_GRID = BATCH // BB


def _detile_body(t_ref, out_ref):
    z = t_ref[...].T
    out_ref[...] = jnp.concatenate([z[: VB // 2], z[VB // 2:]], axis=1)


def _gather_body(x_hbm, oidx_hbm, table_hbm, out_hbm, idx_v, oidx_v, rows_v,
                 g0, g1, g2, g3, s0, s1, s2, s3):
    gs = (g0, g1, g2, g3)
    ss = (s0, s1, s2, s3)
    wid = lax.axis_index("s") * 2 + lax.axis_index("c")
    base = wid * B_PER_W

    def fire_gather(i, s):
        off = base + i * CHUNK
        pltpu.sync_copy(x_hbm.at[pl.ds(off, CHUNK)], idx_v.at[s])
        pltpu.sync_copy(oidx_hbm.at[pl.ds(off, CHUNK)], oidx_v.at[s])
        pltpu.async_copy(table_hbm.at[idx_v.at[s]], rows_v.at[s], gs[s])

    def wait_gather(s):
        pltpu.make_async_copy(
            table_hbm.at[idx_v.at[s]], rows_v.at[s], gs[s]).wait()

    def fire_scatter(s):
        pltpu.async_copy(rows_v.at[s], out_hbm.at[oidx_v.at[s]], ss[s])

    def wait_scatter(s):
        pltpu.make_async_copy(
            rows_v.at[s], out_hbm.at[oidx_v.at[s]], ss[s]).wait()

    def outer(j, carry):
        for s in range(NBUF):
            if s < 2:
                @pl.when(j >= 1)
                def _():
                    wait_scatter(s)
                    fire_gather(NBUF * j + s, s)
                    wait_gather((s + 2) % NBUF)
                    fire_scatter((s + 2) % NBUF)
            else:
                @pl.when(j >= 1)
                def _():
                    wait_scatter(s)
                fire_gather(NBUF * j + s, s)
                wait_gather(s - 2)
                fire_scatter(s - 2)
        return carry

    fire_gather(0, 0)
    fire_gather(1, 1)
    lax.fori_loop(0, N_OUTER, outer, 0)
    wait_gather(2)
    fire_scatter(2)
    wait_gather(3)
    fire_scatter(3)
    for s in range(NBUF):
        wait_scatter(s)


def _retile_body(in_ref, out_ref):
    y = in_ref[...]
    out_ref[...] = y.transpose(0, 2, 1).reshape(SEQ * D_MODEL, BB)


@jax.jit
def kernel(x, table):
    packed = pl.pallas_call(
        _detile_body,
        grid=(A_GRID,),
        in_specs=[pl.BlockSpec((D_MODEL, VB), lambda i: (0, i))],
        out_specs=pl.BlockSpec((VB // 2, 2 * D_MODEL), lambda i: (i, 0)),
        out_shape=jax.ShapeDtypeStruct((A_GRID * VB // 2, 2 * D_MODEL),
                                       jnp.float32),
    )(table.T)
    table_l = packed.reshape(-1).reshape(A_GRID * VB, D_MODEL)

    xf = x.reshape(-1).astype(jnp.int32)
    hb = (VB // 2).bit_length() - 1
    xf = ((xf >> (hb + 1)) << (hb + 1)) + ((xf & (VB // 2 - 1)) << 1) + (
        (xf >> hb) & 1)
    bb = jax.lax.broadcasted_iota(jnp.int32, (BATCH, SEQ), 0)
    tt = jax.lax.broadcasted_iota(jnp.int32, (BATCH, SEQ), 1)
    oidx = (((tt >> 1) * BATCH + bb) * 2 + (tt & 1)).reshape(-1)
    mesh = plsc.VectorSubcoreMesh(core_axis_name="c", subcore_axis_name="s")
    rows = pl.kernel(
        _gather_body,
        out_type=jax.ShapeDtypeStruct((B_TOTAL, D_MODEL), jnp.float32),
        mesh=mesh,
        scratch_types=[
            pltpu.VMEM((NBUF, CHUNK), jnp.int32),
            pltpu.VMEM((NBUF, CHUNK), jnp.int32),
            pltpu.VMEM((NBUF, CHUNK, D_MODEL), jnp.float32),
        ] + [pltpu.SemaphoreType.DMA] * (2 * NBUF),
        compiler_params=pltpu.CompilerParams(use_tc_tiling_on_sc=False),
    )(xf, oidx, table_l)

    out_t = pl.pallas_call(
        _retile_body,
        grid=(C_GRID,),
        in_specs=[pl.BlockSpec((SEQ // 2, BB, 2 * D_MODEL),
                               lambda i: (0, i, 0))],
        out_specs=pl.BlockSpec((SEQ * D_MODEL, BB), lambda i: (0, i)),
        out_shape=jax.ShapeDtypeStruct((SEQ * D_MODEL, BATCH), jnp.float32),
    )(rows.reshape(SEQ // 2, BATCH, 2 * D_MODEL))
    return out_t.reshape(SEQ, D_MODEL, BATCH).transpose(2, 0, 1)
